# Initial kernel scaffold; baseline (speedup 1.0000x reference)
import jax
import jax.numpy as jnp
from jax.experimental import pallas as pl


def kernel(text, embedding_table):
    raise NotImplementedError("write your pallas kernel here")


# speedup vs baseline: 2.3785x; 2.3785x over previous
import functools

import jax
import jax.numpy as jnp
from jax import lax
from jax.experimental import pallas as pl
from jax.experimental.pallas import tpu as pltpu
from jax.experimental.pallas import tpu_sc as plsc

HIDDEN = 512
B_TOTAL = 4096 * 200
NC, NS = 2, 16
NW = NC * NS
B_PER_W = B_TOTAL // NW
CB = 80
NBUF = 2
N_CHUNKS = B_PER_W // CB
N_BLOCKS = N_CHUNKS // NBUF


def _emb_body(table_hbm, idx_hbm, out_hbm, idx_v, rows_v, sem0, sem1):
    sems = (sem0, sem1)
    wid = lax.axis_index("s") * NC + lax.axis_index("c")
    base = wid * B_PER_W

    def load_idx(g, b):
        pltpu.sync_copy(idx_hbm.at[pl.ds(base + g * CB, CB)], idx_v.at[b])

    def start_gather(b):
        pltpu.async_copy(table_hbm.at[idx_v.at[b]], rows_v.at[b], sems[b])

    def wait_gather(b):
        pltpu.make_async_copy(
            table_hbm.at[idx_v.at[b]], rows_v.at[b], sems[b]).wait()

    def store(g, b):
        pltpu.sync_copy(rows_v.at[b], out_hbm.at[pl.ds(base + g * CB, CB)])

    load_idx(0, 0)
    start_gather(0)

    def blk_body(blk, carry):
        for b in range(NBUF):
            g = blk * NBUF + b
            load_idx(g + 1, 1 - b)
            start_gather(1 - b)
            wait_gather(b)
            store(g, b)
        return carry

    lax.fori_loop(0, N_BLOCKS - 1, blk_body, 0)

    g0 = (N_BLOCKS - 1) * NBUF
    load_idx(g0 + 1, 1)
    start_gather(1)
    wait_gather(0)
    store(g0, 0)
    wait_gather(1)
    store(g0 + 1, 1)


_gather_call = functools.partial(
    pl.kernel,
    out_type=jax.ShapeDtypeStruct((B_TOTAL, HIDDEN), jnp.float32),
    mesh=plsc.VectorSubcoreMesh(core_axis_name="c", subcore_axis_name="s"),
    scratch_types=[
        pltpu.VMEM((NBUF, CB), jnp.int32),
        pltpu.VMEM((NBUF, CB, HIDDEN), jnp.float32),
        pltpu.SemaphoreType.DMA,
        pltpu.SemaphoreType.DMA,
    ],
)(_emb_body)


def kernel(text, embedding_table):
    flat_idx = text.reshape(-1).astype(jnp.int32)
    out = _gather_call(embedding_table, flat_idx)
    return out.reshape(text.shape + (embedding_table.shape[-1],))
